# baseline (speedup 1.0000x reference)
import dataclasses
import functools

import jax
import jax.numpy as jnp
from jax import lax
from jax.experimental import pallas as pl
from jax.experimental.pallas import tpu as pltpu
from jax.experimental.pallas import tpu_sc as plsc

_VOCAB = 1000
_EMBED = 128
_LANES = 16
_B = 1024
_L = 50
_NNEG = 5
_N = _B * _L
_NSAMP = _NNEG + 1

_NC = 2
_NS = 16
_NW = _NC * _NS
_TOK_W = _N // _NW
_JP = _EMBED // 2
_GT = _LANES
_GRP = 5
_CT = _GRP * _GT
_NCH = _TOK_W // _CT
_CTXW = _CT * _EMBED
_IDXW = _GRP * _NSAMP * _GT
_SCW = _GRP * _GT * _NSAMP * _LANES // 2
_ROWS = _NW * _NCH * _SCW // 128


def _sc_partials(table_p, ctx_in, idx_in):
    mesh = plsc.VectorSubcoreMesh(core_axis_name="c", subcore_axis_name="s")
    cp = pltpu.CompilerParams()
    if "needs_layout_passes" in pltpu.CompilerParams.__dataclass_fields__:
        cp = dataclasses.replace(cp, needs_layout_passes=False)

    @functools.partial(
        pl.kernel,
        out_type=jax.ShapeDtypeStruct((_NW, _NCH, _SCW), jnp.int32),
        mesh=mesh,
        compiler_params=cp,
        scratch_types=[
            pltpu.VMEM((_VOCAB * _JP,), jnp.int32),
            pltpu.VMEM((_CTXW,), jnp.float32),
            pltpu.VMEM((_CTXW,), jnp.float32),
            pltpu.VMEM((_IDXW,), jnp.int32),
            pltpu.VMEM((_IDXW,), jnp.int32),
            pltpu.VMEM((_SCW,), jnp.int32),
            pltpu.VMEM((_SCW,), jnp.int32),
            pltpu.SemaphoreType.DMA,
            pltpu.SemaphoreType.DMA,
            pltpu.SemaphoreType.DMA,
            pltpu.SemaphoreType.DMA,
            pltpu.SemaphoreType.DMA,
        ],
    )
    def part_kernel(
        table_hbm, ctx_hbm, idx_hbm, out_hbm,
        w_v, ctx_v0, ctx_v1, idx_v0, idx_v1, sco_v0, sco_v1,
        wsem, csem_a, csem_b, ssem_a, ssem_b,
    ):
        wid = lax.axis_index("s") * _NC + lax.axis_index("c")

        pltpu.async_copy(table_hbm, w_v, wsem).wait()
        lane = lax.iota(jnp.int32, _LANES)
        zero_i = jnp.zeros((_LANES,), jnp.int32)

        def in_copies(c, ctx_v, idx_v, sem):
            return [
                pltpu.make_async_copy(ctx_hbm.at[wid, c], ctx_v, sem),
                pltpu.make_async_copy(idx_hbm.at[wid, c], idx_v, sem),
            ]

        def sco_copy(c, sco_v, sem):
            return pltpu.make_async_copy(sco_v, out_hbm.at[wid, c], sem)

        def start(copies):
            for cp_ in copies:
                cp_.start()

        def wait(copies):
            for cp_ in copies:
                cp_.wait()

        def unpack(words):
            lo = plsc.bitcast(lax.shift_left(words, 16), jnp.float32)
            hi = plsc.bitcast(
                lax.bitwise_and(words, jnp.int32(-65536)), jnp.float32
            )
            return lo, hi

        def compute(ctx_v, idx_v, sco_v):
            @pl.loop(0, _GRP)
            def _(g):
                coff = g * (_GT * _EMBED)
                ioff = g * (_NSAMP * _GT)
                soff = g * (_NSAMP * _GT * 8)
                bases = [
                    idx_v[pl.ds(ioff + _LANES * k, _LANES)]
                    for k in range(_NSAMP)
                ]

                @pl.loop(0, _GT, step=2)
                def _(t0):
                  for t in (t0, t0 + 1):
                    mask_t = lane == t
                    cvec = [
                        ctx_v[pl.ds(coff + _EMBED * t + _LANES * m, _LANES)]
                        for m in range(8)
                    ]
                    accs = []
                    for k in range(_NSAMP):
                        r = jnp.sum(jnp.where(mask_t, bases[k], zero_i))
                        rb = r * _JP
                        acc = None
                        for m in range(4):
                            wlo, whi = unpack(w_v[pl.ds(rb + _LANES * m, _LANES)])
                            term = wlo * cvec[m] + whi * cvec[4 + m]
                            acc = term if acc is None else acc + term
                        accs.append(acc if k == 0 else -acc)
                    for p in range(_NSAMP // 2):
                        pk = plsc.pack(
                            accs[2 * p], accs[2 * p + 1],
                            format=plsc.PackFormat.INTERLEAVED,
                        )
                        sco_v[pl.ds(soff + 48 * t + _LANES * p, _LANES)] = (
                            plsc.bitcast(pk, jnp.int32)
                        )

        def run_chunk(c, ctx_v, idx_v, sco_v, csem, ssem, first):
            wait(in_copies(c, ctx_v, idx_v, csem))
            if not first:
                sco_copy(c, sco_v, ssem).wait()
            compute(ctx_v, idx_v, sco_v)
            sco_copy(c, sco_v, ssem).start()

        start(in_copies(0, ctx_v0, idx_v0, csem_a))
        start(in_copies(1, ctx_v1, idx_v1, csem_b))
        run_chunk(0, ctx_v0, idx_v0, sco_v0, csem_a, ssem_a, True)
        start(in_copies(2, ctx_v0, idx_v0, csem_a))
        run_chunk(1, ctx_v1, idx_v1, sco_v1, csem_b, ssem_b, True)
        start(in_copies(3, ctx_v1, idx_v1, csem_b))

        @pl.loop(2, _NCH - 2, step=2)
        def _(c):
            run_chunk(c, ctx_v0, idx_v0, sco_v0, csem_a, ssem_a, False)
            start(in_copies(c + 2, ctx_v0, idx_v0, csem_a))
            run_chunk(c + 1, ctx_v1, idx_v1, sco_v1, csem_b, ssem_b, False)
            start(in_copies(c + 3, ctx_v1, idx_v1, csem_b))

        ct = _NCH - 2
        run_chunk(ct, ctx_v0, idx_v0, sco_v0, csem_a, ssem_a, False)
        run_chunk(ct + 1, ctx_v1, idx_v1, sco_v1, csem_b, ssem_b, False)
        sco_copy(ct, sco_v0, ssem_a).wait()
        sco_copy(ct + 1, sco_v1, ssem_b).wait()

    return part_kernel(table_p, ctx_in, idx_in)


def _logsig(x):
    return jnp.minimum(x, 0.0) - jnp.log1p(jnp.exp(-jnp.abs(x)))


_TR = 1920


def _tc_loss(parts):
    def body(p_ref, o_ref):
        i = pl.program_id(0)
        x = p_ref[...]
        lo = lax.bitcast_convert_type(lax.shift_left(x, 16), jnp.float32)
        hi = lax.bitcast_convert_type(
            lax.bitwise_and(x, jnp.int32(-65536)), jnp.float32
        )
        lane = lax.broadcasted_iota(jnp.int32, (_EMBED, 8), 0)
        grp = lax.broadcasted_iota(jnp.int32, (_EMBED, 8), 1)
        msum = jnp.where(lane // _LANES == grp, 1.0, 0.0)
        s_lo = jnp.dot(lo, msum, preferred_element_type=jnp.float32)
        s_hi = jnp.dot(hi, msum, preferred_element_type=jnp.float32)
        blk = jnp.sum(_logsig(s_lo)) + jnp.sum(_logsig(s_hi))

        @pl.when(i == 0)
        def _():
            o_ref[0, 0] = 0.0

        o_ref[0, 0] += -blk

    out = pl.pallas_call(
        body,
        grid=(_ROWS // _TR,),
        in_specs=[pl.BlockSpec((_TR, _EMBED), lambda i: (i, 0))],
        out_specs=pl.BlockSpec(memory_space=pltpu.SMEM),
        out_shape=jax.ShapeDtypeStruct((1, 1), jnp.float32),
    )(parts)
    return out[0, 0]


def kernel(sentence, context, W, neg_samples):
    Wb = W.astype(jnp.bfloat16)
    table_p = lax.bitcast_convert_type(
        jnp.stack([Wb[:, :_JP], Wb[:, _JP:]], axis=-1), jnp.int32
    ).reshape(_VOCAB * _JP)
    ctx_in = context.reshape(_NW, _NCH, _CTXW)
    idx6 = jnp.concatenate(
        [sentence.reshape(1, _N), neg_samples.reshape(_N, _NNEG).T], axis=0
    )
    idx_in = (
        idx6.reshape(_NSAMP, _NW, _TOK_W // _GT, _GT)
        .transpose(1, 2, 0, 3)
        .reshape(_NW, _NCH, _IDXW)
        .astype(jnp.int32)
    )
    parts = _sc_partials(table_p, ctx_in, idx_in)
    return _tc_loss(parts.reshape(_ROWS, _EMBED))


# speedup vs baseline: 1.0205x; 1.0205x over previous
import dataclasses
import functools

import jax
import jax.numpy as jnp
from jax import lax
from jax.experimental import pallas as pl
from jax.experimental.pallas import tpu as pltpu
from jax.experimental.pallas import tpu_sc as plsc

_VOCAB = 1000
_EMBED = 128
_LANES = 16
_B = 1024
_L = 50
_NNEG = 5
_N = _B * _L
_NSAMP = _NNEG + 1

_NC = 2
_NS = 16
_NW = _NC * _NS
_TOK_W = _N // _NW
_JP = _EMBED // 2
_GT = _LANES
_GRP = 5
_CT = _GRP * _GT
_NCH = _TOK_W // _CT
_CTXW = _CT * _EMBED
_IDXW = _GRP * _NSAMP * _GT
_SCW = _GRP * _GT * _NSAMP * _LANES // 2
_ROWS = _NW * _NCH * _SCW // 128


def _sc_partials(table_p, ctx_in, idx_in):
    mesh = plsc.VectorSubcoreMesh(core_axis_name="c", subcore_axis_name="s")
    cp = pltpu.CompilerParams()
    if "needs_layout_passes" in pltpu.CompilerParams.__dataclass_fields__:
        cp = dataclasses.replace(cp, needs_layout_passes=False)

    @functools.partial(
        pl.kernel,
        out_type=jax.ShapeDtypeStruct((_NW, _NCH, _SCW), jnp.int32),
        mesh=mesh,
        compiler_params=cp,
        scratch_types=[
            pltpu.VMEM((_VOCAB * _JP,), jnp.int32),
            pltpu.VMEM((_CT, _EMBED), jnp.float32),
            pltpu.VMEM((_CT, _EMBED), jnp.float32),
            pltpu.VMEM((_IDXW,), jnp.int32),
            pltpu.VMEM((_IDXW,), jnp.int32),
            pltpu.VMEM((_SCW,), jnp.int32),
            pltpu.VMEM((_SCW,), jnp.int32),
            pltpu.SemaphoreType.DMA,
            pltpu.SemaphoreType.DMA,
            pltpu.SemaphoreType.DMA,
            pltpu.SemaphoreType.DMA,
            pltpu.SemaphoreType.DMA,
        ],
    )
    def part_kernel(
        table_hbm, ctx_hbm, idx_hbm, out_hbm,
        w_v, ctx_v0, ctx_v1, idx_v0, idx_v1, sco_v0, sco_v1,
        wsem, csem_a, csem_b, ssem_a, ssem_b,
    ):
        wid = lax.axis_index("s") * _NC + lax.axis_index("c")

        pltpu.async_copy(table_hbm, w_v, wsem).wait()
        lane = lax.iota(jnp.int32, _LANES)
        zero_i = jnp.zeros((_LANES,), jnp.int32)

        def in_copies(c, ctx_v, idx_v, sem):
            return [
                pltpu.make_async_copy(
                    ctx_hbm.at[pl.ds((wid * _NCH + c) * _CT, _CT)], ctx_v, sem
                ),
                pltpu.make_async_copy(idx_hbm.at[wid, c], idx_v, sem),
            ]

        def sco_copy(c, sco_v, sem):
            return pltpu.make_async_copy(sco_v, out_hbm.at[wid, c], sem)

        def start(copies):
            for cp_ in copies:
                cp_.start()

        def wait(copies):
            for cp_ in copies:
                cp_.wait()

        def unpack(words):
            lo = plsc.bitcast(lax.shift_left(words, 16), jnp.float32)
            hi = plsc.bitcast(
                lax.bitwise_and(words, jnp.int32(-65536)), jnp.float32
            )
            return lo, hi

        def compute(ctx_v, idx_v, sco_v):
            @pl.loop(0, _GRP)
            def _(g):
                coff = g * (_GT * _EMBED)
                ioff = g * (_NSAMP * _GT)
                soff = g * (_NSAMP * _GT * 8)
                bases = [
                    idx_v[pl.ds(ioff + _LANES * k, _LANES)]
                    for k in range(_NSAMP)
                ]

                @pl.loop(0, _GT)
                def _(t):
                    mask_t = lane == t
                    tau = g * _GT + t
                    cvec = [
                        ctx_v[tau, pl.ds(_LANES * m, _LANES)]
                        for m in range(8)
                    ]
                    accs = []
                    for k in range(_NSAMP):
                        r = jnp.sum(jnp.where(mask_t, bases[k], zero_i))
                        rb = r * _JP
                        acc = None
                        for m in range(4):
                            wlo, whi = unpack(w_v[pl.ds(rb + _LANES * m, _LANES)])
                            term = wlo * cvec[m] + whi * cvec[4 + m]
                            acc = term if acc is None else acc + term
                        accs.append(acc if k == 0 else -acc)
                    for p in range(_NSAMP // 2):
                        pk = plsc.pack(
                            accs[2 * p], accs[2 * p + 1],
                            format=plsc.PackFormat.INTERLEAVED,
                        )
                        sco_v[pl.ds(soff + 48 * t + _LANES * p, _LANES)] = (
                            plsc.bitcast(pk, jnp.int32)
                        )

        def run_chunk(c, ctx_v, idx_v, sco_v, csem, ssem, first):
            wait(in_copies(c, ctx_v, idx_v, csem))
            if not first:
                sco_copy(c, sco_v, ssem).wait()
            compute(ctx_v, idx_v, sco_v)
            sco_copy(c, sco_v, ssem).start()

        start(in_copies(0, ctx_v0, idx_v0, csem_a))
        start(in_copies(1, ctx_v1, idx_v1, csem_b))
        run_chunk(0, ctx_v0, idx_v0, sco_v0, csem_a, ssem_a, True)
        start(in_copies(2, ctx_v0, idx_v0, csem_a))
        run_chunk(1, ctx_v1, idx_v1, sco_v1, csem_b, ssem_b, True)
        start(in_copies(3, ctx_v1, idx_v1, csem_b))

        @pl.loop(2, _NCH - 2, step=2)
        def _(c):
            run_chunk(c, ctx_v0, idx_v0, sco_v0, csem_a, ssem_a, False)
            start(in_copies(c + 2, ctx_v0, idx_v0, csem_a))
            run_chunk(c + 1, ctx_v1, idx_v1, sco_v1, csem_b, ssem_b, False)
            start(in_copies(c + 3, ctx_v1, idx_v1, csem_b))

        ct = _NCH - 2
        run_chunk(ct, ctx_v0, idx_v0, sco_v0, csem_a, ssem_a, False)
        run_chunk(ct + 1, ctx_v1, idx_v1, sco_v1, csem_b, ssem_b, False)
        sco_copy(ct, sco_v0, ssem_a).wait()
        sco_copy(ct + 1, sco_v1, ssem_b).wait()

    return part_kernel(table_p, ctx_in, idx_in)


def _logsig(x):
    return jnp.minimum(x, 0.0) - jnp.log1p(jnp.exp(-jnp.abs(x)))


_TR = 1920


def _tc_loss(parts):
    def body(p_ref, o_ref):
        i = pl.program_id(0)
        x = p_ref[...]
        lo = lax.bitcast_convert_type(lax.shift_left(x, 16), jnp.float32)
        hi = lax.bitcast_convert_type(
            lax.bitwise_and(x, jnp.int32(-65536)), jnp.float32
        )
        lane = lax.broadcasted_iota(jnp.int32, (_EMBED, 8), 0)
        grp = lax.broadcasted_iota(jnp.int32, (_EMBED, 8), 1)
        msum = jnp.where(lane // _LANES == grp, 1.0, 0.0)
        s_lo = jnp.dot(lo, msum, preferred_element_type=jnp.float32)
        s_hi = jnp.dot(hi, msum, preferred_element_type=jnp.float32)
        blk = jnp.sum(_logsig(s_lo)) + jnp.sum(_logsig(s_hi))

        @pl.when(i == 0)
        def _():
            o_ref[0, 0] = 0.0

        o_ref[0, 0] += -blk

    out = pl.pallas_call(
        body,
        grid=(_ROWS // _TR,),
        in_specs=[pl.BlockSpec((_TR, _EMBED), lambda i: (i, 0))],
        out_specs=pl.BlockSpec(memory_space=pltpu.SMEM),
        out_shape=jax.ShapeDtypeStruct((1, 1), jnp.float32),
    )(parts)
    return out[0, 0]


def kernel(sentence, context, W, neg_samples):
    Wb = W.astype(jnp.bfloat16)
    table_p = lax.bitcast_convert_type(
        jnp.stack([Wb[:, :_JP], Wb[:, _JP:]], axis=-1), jnp.int32
    ).reshape(_VOCAB * _JP)
    ctx_in = context.reshape(_N, _EMBED)
    idx6 = jnp.concatenate(
        [sentence.reshape(1, _N), neg_samples.reshape(_N, _NNEG).T], axis=0
    )
    idx_in = (
        idx6.reshape(_NSAMP, _NW, _TOK_W // _GT, _GT)
        .transpose(1, 2, 0, 3)
        .reshape(_NW, _NCH, _IDXW)
        .astype(jnp.int32)
    )
    parts = _sc_partials(table_p, ctx_in, idx_in)
    return _tc_loss(parts.reshape(_ROWS, _EMBED))


# speedup vs baseline: 1.0848x; 1.0630x over previous
import dataclasses
import functools

import jax
import jax.numpy as jnp
from jax import lax
from jax.experimental import pallas as pl
from jax.experimental.pallas import tpu as pltpu
from jax.experimental.pallas import tpu_sc as plsc

_VOCAB = 1000
_EMBED = 128
_LANES = 16
_B = 1024
_L = 50
_NNEG = 5
_N = _B * _L
_NSAMP = _NNEG + 1

_NC = 2
_NS = 16
_NW = _NC * _NS
_TOK_W = _N // _NW
_JP = _EMBED // 2
_GT = _LANES
_GRP = 5
_CT = _GRP * _GT
_NCH = _TOK_W // _CT
_CTXW = _CT * _EMBED
_IDXW = _GRP * _NSAMP * _GT
_SCW = _GRP * _GT * _NSAMP * _LANES // 2
_SCWP = 4096
_ROWS = _NW * _NCH * _SCWP // 128


def _sc_partials(table_p, ctx_in, idx_in):
    mesh = plsc.VectorSubcoreMesh(core_axis_name="c", subcore_axis_name="s")
    cp = pltpu.CompilerParams()
    if "needs_layout_passes" in pltpu.CompilerParams.__dataclass_fields__:
        cp = dataclasses.replace(cp, needs_layout_passes=False)

    @functools.partial(
        pl.kernel,
        out_type=jax.ShapeDtypeStruct((_NW * _NCH * _SCWP,), jnp.int32),
        mesh=mesh,
        compiler_params=cp,
        scratch_types=[
            pltpu.VMEM((_VOCAB * _JP,), jnp.int32),
            pltpu.VMEM((_CTXW,), jnp.float32),
            pltpu.VMEM((_CTXW,), jnp.float32),
            pltpu.VMEM((_IDXW,), jnp.int32),
            pltpu.VMEM((_IDXW,), jnp.int32),
            pltpu.VMEM((_SCWP,), jnp.int32),
            pltpu.VMEM((_SCWP,), jnp.int32),
            pltpu.SemaphoreType.DMA,
            pltpu.SemaphoreType.DMA,
            pltpu.SemaphoreType.DMA,
            pltpu.SemaphoreType.DMA,
            pltpu.SemaphoreType.DMA,
        ],
    )
    def part_kernel(
        table_hbm, ctx_hbm, idx_hbm, out_hbm,
        w_v, ctx_v0, ctx_v1, idx_v0, idx_v1, sco_v0, sco_v1,
        wsem, csem_a, csem_b, ssem_a, ssem_b,
    ):
        wid = lax.axis_index("s") * _NC + lax.axis_index("c")

        pltpu.async_copy(table_hbm, w_v, wsem).wait()
        lane = lax.iota(jnp.int32, _LANES)
        zero_i = jnp.zeros((_LANES,), jnp.int32)
        for sco_v in (sco_v0, sco_v1):
            for q in range(_SCW, _SCWP, _LANES):
                sco_v[pl.ds(q, _LANES)] = zero_i

        def in_copies(c, ctx_v, idx_v, sem):
            return [
                pltpu.make_async_copy(ctx_hbm.at[wid, c], ctx_v, sem),
                pltpu.make_async_copy(idx_hbm.at[wid, c], idx_v, sem),
            ]

        def sco_copy(c, sco_v, sem):
            return pltpu.make_async_copy(
                sco_v,
                out_hbm.at[pl.ds((wid * _NCH + c) * _SCWP, _SCWP)],
                sem,
            )

        def start(copies):
            for cp_ in copies:
                cp_.start()

        def wait(copies):
            for cp_ in copies:
                cp_.wait()

        def unpack(words):
            lo = plsc.bitcast(lax.shift_left(words, 16), jnp.float32)
            hi = plsc.bitcast(
                lax.bitwise_and(words, jnp.int32(-65536)), jnp.float32
            )
            return lo, hi

        def compute(ctx_v, idx_v, sco_v):
            @pl.loop(0, _GRP)
            def _(g):
                coff = g * (_GT * _EMBED)
                ioff = g * (_NSAMP * _GT)
                soff = g * (_NSAMP * _GT * 8)
                bases = [
                    idx_v[pl.ds(ioff + _LANES * k, _LANES)]
                    for k in range(_NSAMP)
                ]

                @pl.loop(0, _GT)
                def _(t):
                    mask_t = lane == t
                    cvec = [
                        ctx_v[pl.ds(coff + _EMBED * t + _LANES * m, _LANES)]
                        for m in range(8)
                    ]
                    accs = []
                    for k in range(_NSAMP):
                        r = jnp.sum(jnp.where(mask_t, bases[k], zero_i))
                        rb = r * _JP
                        acc = None
                        for m in range(4):
                            wlo, whi = unpack(w_v[pl.ds(rb + _LANES * m, _LANES)])
                            term = wlo * cvec[m] + whi * cvec[4 + m]
                            acc = term if acc is None else acc + term
                        accs.append(acc if k == 0 else -acc)
                    for p in range(_NSAMP // 2):
                        pk = plsc.pack(
                            accs[2 * p], accs[2 * p + 1],
                            format=plsc.PackFormat.INTERLEAVED,
                        )
                        sco_v[pl.ds(soff + 48 * t + _LANES * p, _LANES)] = (
                            plsc.bitcast(pk, jnp.int32)
                        )

        def run_chunk(c, ctx_v, idx_v, sco_v, csem, ssem, first):
            wait(in_copies(c, ctx_v, idx_v, csem))
            if not first:
                sco_copy(c, sco_v, ssem).wait()
            compute(ctx_v, idx_v, sco_v)
            sco_copy(c, sco_v, ssem).start()

        start(in_copies(0, ctx_v0, idx_v0, csem_a))
        start(in_copies(1, ctx_v1, idx_v1, csem_b))
        run_chunk(0, ctx_v0, idx_v0, sco_v0, csem_a, ssem_a, True)
        start(in_copies(2, ctx_v0, idx_v0, csem_a))
        run_chunk(1, ctx_v1, idx_v1, sco_v1, csem_b, ssem_b, True)
        start(in_copies(3, ctx_v1, idx_v1, csem_b))

        @pl.loop(2, _NCH - 2, step=2)
        def _(c):
            run_chunk(c, ctx_v0, idx_v0, sco_v0, csem_a, ssem_a, False)
            start(in_copies(c + 2, ctx_v0, idx_v0, csem_a))
            run_chunk(c + 1, ctx_v1, idx_v1, sco_v1, csem_b, ssem_b, False)
            start(in_copies(c + 3, ctx_v1, idx_v1, csem_b))

        ct = _NCH - 2
        run_chunk(ct, ctx_v0, idx_v0, sco_v0, csem_a, ssem_a, False)
        run_chunk(ct + 1, ctx_v1, idx_v1, sco_v1, csem_b, ssem_b, False)
        sco_copy(ct, sco_v0, ssem_a).wait()
        sco_copy(ct + 1, sco_v1, ssem_b).wait()

    return part_kernel(table_p, ctx_in, idx_in)


def _logsig(x):
    return jnp.minimum(x, 0.0) - jnp.log1p(jnp.exp(-jnp.abs(x)))


_TR = 2048


def _tc_loss(parts):
    def body(p_ref, o_ref):
        i = pl.program_id(0)
        x = p_ref[...]
        lo = lax.bitcast_convert_type(lax.shift_left(x, 16), jnp.float32)
        hi = lax.bitcast_convert_type(
            lax.bitwise_and(x, jnp.int32(-65536)), jnp.float32
        )
        lane = lax.broadcasted_iota(jnp.int32, (_EMBED, 8), 0)
        grp = lax.broadcasted_iota(jnp.int32, (_EMBED, 8), 1)
        msum = jnp.where(lane // _LANES == grp, 1.0, 0.0)
        s_lo = jnp.dot(lo, msum, preferred_element_type=jnp.float32)
        s_hi = jnp.dot(hi, msum, preferred_element_type=jnp.float32)
        rowl = lax.broadcasted_iota(jnp.int32, (_TR, 8), 0)
        valid = (rowl % 32) < 30
        blk = jnp.sum(jnp.where(valid, _logsig(s_lo) + _logsig(s_hi), 0.0))

        @pl.when(i == 0)
        def _():
            o_ref[0, 0] = 0.0

        o_ref[0, 0] += -blk

    out = pl.pallas_call(
        body,
        grid=(_ROWS // _TR,),
        in_specs=[pl.BlockSpec((_TR, _EMBED), lambda i: (i, 0))],
        out_specs=pl.BlockSpec(memory_space=pltpu.SMEM),
        out_shape=jax.ShapeDtypeStruct((1, 1), jnp.float32),
    )(parts)
    return out[0, 0]


def kernel(sentence, context, W, neg_samples):
    Wb = W.astype(jnp.bfloat16)
    table_p = lax.bitcast_convert_type(
        jnp.stack([Wb[:, :_JP], Wb[:, _JP:]], axis=-1), jnp.int32
    ).reshape(_VOCAB * _JP)
    ctx_in = context.reshape(_NW, _NCH, _CTXW)
    idx6 = jnp.concatenate(
        [sentence.reshape(1, _N), neg_samples.reshape(_N, _NNEG).T], axis=0
    )
    idx_in = (
        idx6.reshape(_NSAMP, _NW, _TOK_W // _GT, _GT)
        .transpose(1, 2, 0, 3)
        .reshape(_NW, _NCH, _IDXW)
        .astype(jnp.int32)
    )
    parts = _sc_partials(table_p, ctx_in, idx_in)
    return _tc_loss(parts.reshape(_ROWS, _EMBED))
